# baseline (speedup 1.0000x reference)
import functools

import jax
import jax.numpy as jnp
from jax import lax
from jax.experimental import pallas as pl
from jax.experimental.pallas import tpu as pltpu
from jax.experimental.pallas import tpu_sc as plsc

_NBUCK = 100000
_NHASH = 4
_SUBD = 16
_EMBD = 64
_PACK = 8
_LANE = _PACK * _SUBD
_NCORE = 2
_NSUB = 16
_NW = _NCORE * _NSUB
_CHUNK = 1024


def _mod_nbuck(t):
    nb = jnp.int32(_NBUCK)
    q = (t.astype(jnp.float32) * jnp.float32(1.0 / _NBUCK)).astype(jnp.int32)
    r = t - q * nb
    r = jnp.where(r < 0, r + nb, r)
    r = jnp.where(r >= nb, r - nb, r)
    return r


def _sc_body(nflat, ids_hbm, params_hbm, tbl_hbm, out_hbm,
             params_v, ids2_v, idx4_v, gath4_v, sem_i, sem_g, sem_o):
    wid = lax.axis_index("s") * _NCORE + lax.axis_index("c")
    tpw = nflat // _NW
    nchunks = tpw // _CHUNK
    pltpu.sync_copy(params_hbm, params_v)

    def ids_slice(k):
        base = pl.multiple_of(
            wid * jnp.int32(tpw) + k * jnp.int32(_CHUNK), _CHUNK)
        return ids_hbm.at[pl.ds(base, _CHUNK)]

    def out_drains(base):
        for i in range(_NHASH):
            pltpu.make_async_copy(
                gath4_v.at[jnp.int32(i)],
                out_hbm.at[jnp.int32(i), pl.ds(base, _CHUNK), :],
                sem_o).wait()

    pltpu.async_copy(ids_slice(jnp.int32(0)), ids2_v.at[jnp.int32(0)], sem_i)

    def chunk_body(k, carry):
        base = pl.multiple_of(
            wid * jnp.int32(tpw) + k * jnp.int32(_CHUNK), _CHUNK)
        buf = lax.rem(k, jnp.int32(2))
        pltpu.make_async_copy(ids_slice(k), ids2_v.at[buf], sem_i).wait()

        @pl.when(k + jnp.int32(1) < jnp.int32(nchunks))
        def _prefetch():
            pltpu.async_copy(ids_slice(k + jnp.int32(1)),
                             ids2_v.at[jnp.int32(1) - buf], sem_i)

        for i in range(_NHASH):
            am = _mod_nbuck(params_v[i, 0, :])
            cm = _mod_nbuck(am * jnp.int32(1024))
            bm = _mod_nbuck(params_v[i, 1, :])

            def hash_body(j, carry2):
                x = ids2_v[buf, pl.ds(j * jnp.int32(16), 16)]
                xh = lax.shift_right_logical(x, jnp.int32(10))
                xl = lax.bitwise_and(x, jnp.int32(1023))
                idx4_v[jnp.int32(i), pl.ds(j * jnp.int32(16), 16)] = (
                    _mod_nbuck(xh * cm + xl * am + bm))
                return carry2

            lax.fori_loop(jnp.int32(0), jnp.int32(_CHUNK // 16), hash_body,
                          jnp.int32(0))

        @pl.when(k > jnp.int32(0))
        def _drain_prev():
            out_drains(pl.multiple_of(base - jnp.int32(_CHUNK), _CHUNK))

        gathers = []
        for i in range(_NHASH):
            gathers.append(pltpu.async_copy(
                tbl_hbm.at[jnp.int32(i)].at[idx4_v.at[jnp.int32(i)]],
                gath4_v.at[jnp.int32(i)], sem_g))
        for i in range(_NHASH):
            gathers[i].wait()
            pltpu.async_copy(
                gath4_v.at[jnp.int32(i)],
                out_hbm.at[jnp.int32(i), pl.ds(base, _CHUNK), :], sem_o)
        return carry

    lax.fori_loop(jnp.int32(0), jnp.int32(nchunks), chunk_body, jnp.int32(0))
    last = pl.multiple_of(
        wid * jnp.int32(tpw) + jnp.int32((nchunks - 1) * _CHUNK), _CHUNK)
    out_drains(last)


@functools.lru_cache(maxsize=None)
def _make_sc_gather(nflat):
    mesh = plsc.VectorSubcoreMesh(core_axis_name="c", subcore_axis_name="s")
    return pl.kernel(
        functools.partial(_sc_body, nflat),
        out_type=jax.ShapeDtypeStruct((_NHASH, nflat, _SUBD), jnp.float32),
        mesh=mesh,
        scratch_types=[
            pltpu.VMEM((_NHASH, 2, 16), jnp.int32),
            pltpu.VMEM((2, _CHUNK), jnp.int32),
            pltpu.VMEM((_NHASH, _CHUNK), jnp.int32),
            pltpu.VMEM((_NHASH, _CHUNK, _SUBD), jnp.float32),
            pltpu.SemaphoreType.DMA,
            pltpu.SemaphoreType.DMA,
            pltpu.SemaphoreType.DMA,
        ],
        compiler_params=pltpu.CompilerParams(use_tc_tiling_on_sc=False),
    )


def _mix_body(g_ref, w_ref, b_ref, o_ref):
    acc = jnp.dot(g_ref[0], w_ref[0], preferred_element_type=jnp.float32)
    for i in range(1, _NHASH):
        acc = acc + jnp.dot(g_ref[i], w_ref[i],
                            preferred_element_type=jnp.float32)
    o_ref[:, :] = acc + b_ref[0:1, :]


@functools.lru_cache(maxsize=None)
def _make_mix(npack, rows):
    return pl.pallas_call(
        _mix_body,
        grid=(npack // rows,),
        in_specs=[
            pl.BlockSpec((_NHASH, rows, _LANE),
                         lambda r: (jnp.int32(0), r, jnp.int32(0))),
            pl.BlockSpec((_NHASH, _LANE, _PACK * _EMBD),
                         lambda r: (jnp.int32(0), jnp.int32(0), jnp.int32(0))),
            pl.BlockSpec((1, _PACK * _EMBD),
                         lambda r: (jnp.int32(0), jnp.int32(0))),
        ],
        out_specs=pl.BlockSpec((rows, _PACK * _EMBD),
                               lambda r: (r, jnp.int32(0))),
        out_shape=jax.ShapeDtypeStruct((npack, _PACK * _EMBD), jnp.float32),
    )


def kernel(input_ids, table0, table1, table2, table3, W, b, hash_a, hash_b):
    bsz, seq = input_ids.shape
    nflat = bsz * seq
    flat = input_ids.reshape(-1).astype(jnp.int32)
    params = jnp.stack(
        [hash_a.astype(jnp.int32), hash_b.astype(jnp.int32)], axis=1)
    params = jnp.broadcast_to(params[:, :, None], (_NHASH, 2, 16))

    tbl = jnp.stack([table0, table1, table2, table3])
    g = _make_sc_gather(nflat)(flat, params, tbl)
    g = g.reshape(_NHASH, nflat // _PACK, _LANE)

    w4 = W.astype(jnp.float32).T.reshape(_NHASH, _SUBD, _EMBD)
    eye = jnp.eye(_PACK, dtype=jnp.float32)
    wk = jax.vmap(lambda wi: jnp.kron(eye, wi))(w4)
    bp = jnp.tile(b.astype(jnp.float32), _PACK).reshape(1, _PACK * _EMBD)

    packed = _make_mix(nflat // _PACK, 512)(g, wk, bp)
    out_dtype = jnp.result_type(table0.dtype, W.dtype, b.dtype)
    return packed.astype(out_dtype).reshape(bsz, seq, _EMBD)


# speedup vs baseline: 1.0032x; 1.0032x over previous
import functools

import jax
import jax.numpy as jnp
from jax import lax
from jax.experimental import pallas as pl
from jax.experimental.pallas import tpu as pltpu
from jax.experimental.pallas import tpu_sc as plsc

_NBUCK = 100000
_NHASH = 4
_SUBD = 16
_EMBD = 64
_PACK = 8
_LANE = _PACK * _SUBD
_NCORE = 2
_NSUB = 16
_NW = _NCORE * _NSUB
_CHUNK = 1024


def _mod_nbuck(t):
    nb = jnp.int32(_NBUCK)
    q = (t.astype(jnp.float32) * jnp.float32(1.0 / _NBUCK)).astype(jnp.int32)
    r = t - q * nb
    r = jnp.where(r < 0, r + nb, r)
    r = jnp.where(r >= nb, r - nb, r)
    return r


def _sc_body(nflat, ids_hbm, params_hbm, t0, t1, t2, t3, out_hbm,
             params_v, ids_v, idx4_v, gath4_v, sem_g, sem_o):
    tables = (t0, t1, t2, t3)
    wid = lax.axis_index("s") * _NCORE + lax.axis_index("c")
    tpw = nflat // _NW
    pltpu.sync_copy(params_hbm, params_v)

    def chunk_body(k, carry):
        base = pl.multiple_of(
            wid * jnp.int32(tpw) + k * jnp.int32(_CHUNK), _CHUNK)
        pltpu.sync_copy(ids_hbm.at[pl.ds(base, _CHUNK)], ids_v)
        gathers = []
        for i in range(_NHASH):
            am = _mod_nbuck(params_v[i, 0, :])
            cm = _mod_nbuck(am * jnp.int32(1024))
            bm = _mod_nbuck(params_v[i, 1, :])

            def hash_body(j, carry2):
                x = ids_v[pl.ds(j * jnp.int32(16), 16)]
                xh = lax.shift_right_logical(x, jnp.int32(10))
                xl = lax.bitwise_and(x, jnp.int32(1023))
                idx4_v[jnp.int32(i), pl.ds(j * jnp.int32(16), 16)] = (
                    _mod_nbuck(xh * cm + xl * am + bm))
                return carry2

            lax.fori_loop(jnp.int32(0), jnp.int32(_CHUNK // 16), hash_body,
                          jnp.int32(0))
            gathers.append(pltpu.async_copy(
                tables[i].at[idx4_v.at[jnp.int32(i)]],
                gath4_v.at[jnp.int32(i)], sem_g))
        outs = []
        for i in range(_NHASH):
            gathers[i].wait()
            outs.append(pltpu.async_copy(
                gath4_v.at[jnp.int32(i)],
                out_hbm.at[jnp.int32(i), pl.ds(base, _CHUNK), :], sem_o))
        for cp in outs:
            cp.wait()
        return carry

    lax.fori_loop(jnp.int32(0), jnp.int32(tpw // _CHUNK), chunk_body,
                  jnp.int32(0))


@functools.lru_cache(maxsize=None)
def _make_sc_gather(nflat):
    mesh = plsc.VectorSubcoreMesh(core_axis_name="c", subcore_axis_name="s")
    return pl.kernel(
        functools.partial(_sc_body, nflat),
        out_type=jax.ShapeDtypeStruct((_NHASH, nflat, _SUBD), jnp.float32),
        mesh=mesh,
        scratch_types=[
            pltpu.VMEM((_NHASH, 2, 16), jnp.int32),
            pltpu.VMEM((_CHUNK,), jnp.int32),
            pltpu.VMEM((_NHASH, _CHUNK), jnp.int32),
            pltpu.VMEM((_NHASH, _CHUNK, _SUBD), jnp.float32),
            pltpu.SemaphoreType.DMA,
            pltpu.SemaphoreType.DMA,
        ],
        compiler_params=pltpu.CompilerParams(use_tc_tiling_on_sc=False),
    )


def _mix_body(g_ref, w_ref, b_ref, o_ref):
    acc = jnp.dot(g_ref[0], w_ref[0], preferred_element_type=jnp.float32)
    for i in range(1, _NHASH):
        acc = acc + jnp.dot(g_ref[i], w_ref[i],
                            preferred_element_type=jnp.float32)
    o_ref[:, :] = acc + b_ref[0:1, :]


@functools.lru_cache(maxsize=None)
def _make_mix(npack, rows):
    return pl.pallas_call(
        _mix_body,
        grid=(npack // rows,),
        in_specs=[
            pl.BlockSpec((_NHASH, rows, _LANE),
                         lambda r: (jnp.int32(0), r, jnp.int32(0))),
            pl.BlockSpec((_NHASH, _LANE, _PACK * _EMBD),
                         lambda r: (jnp.int32(0), jnp.int32(0), jnp.int32(0))),
            pl.BlockSpec((1, _PACK * _EMBD),
                         lambda r: (jnp.int32(0), jnp.int32(0))),
        ],
        out_specs=pl.BlockSpec((rows, _PACK * _EMBD),
                               lambda r: (r, jnp.int32(0))),
        out_shape=jax.ShapeDtypeStruct((npack, _PACK * _EMBD), jnp.float32),
    )


def kernel(input_ids, table0, table1, table2, table3, W, b, hash_a, hash_b):
    bsz, seq = input_ids.shape
    nflat = bsz * seq
    flat = input_ids.reshape(-1).astype(jnp.int32)
    params = jnp.stack(
        [hash_a.astype(jnp.int32), hash_b.astype(jnp.int32)], axis=1)
    params = jnp.broadcast_to(params[:, :, None], (_NHASH, 2, 16))

    w4 = W.astype(jnp.float32).T.reshape(_NHASH, _SUBD, _EMBD)
    eye = jnp.eye(_PACK, dtype=jnp.float32)
    wk = jax.vmap(lambda wi: jnp.kron(eye, wi))(w4)
    bp = jnp.tile(b.astype(jnp.float32), _PACK).reshape(1, _PACK * _EMBD)

    half = nflat // 2
    packs = []
    for h in range(2):
        fh = lax.slice_in_dim(flat, h * half, (h + 1) * half)
        gh = _make_sc_gather(half)(fh, params, table0, table1, table2, table3)
        gh = gh.reshape(_NHASH, half // _PACK, _LANE)
        packs.append(_make_mix(half // _PACK, 512)(gh, wk, bp))
    packed = jnp.concatenate(packs, axis=0)
    out_dtype = jnp.result_type(table0.dtype, W.dtype, b.dtype)
    return packed.astype(out_dtype).reshape(bsz, seq, _EMBD)


# speedup vs baseline: 1.0186x; 1.0154x over previous
import functools

import jax
import jax.numpy as jnp
from jax import lax
from jax.experimental import pallas as pl
from jax.experimental.pallas import tpu as pltpu
from jax.experimental.pallas import tpu_sc as plsc

_NBUCK = 100000
_NHASH = 4
_SUBD = 16
_EMBD = 64
_PACK = 8
_LANE = _PACK * _SUBD
_NCORE = 2
_NSUB = 16
_NW = _NCORE * _NSUB
_CHUNK = 1024


def _mod_nbuck(t):
    nb = jnp.int32(_NBUCK)
    q = (t.astype(jnp.float32) * jnp.float32(1.0 / _NBUCK)).astype(jnp.int32)
    r = t - q * nb
    r = jnp.where(r < 0, r + nb, r)
    r = jnp.where(r >= nb, r - nb, r)
    return r


def _sc_body(nflat, ids_hbm, params_hbm, t0, t1, t2, t3, out_hbm,
             params_v, ids_v, idx4_v, gath4_v, sem_g, sem_o):
    tables = (t0, t1, t2, t3)
    wid = lax.axis_index("s") * _NCORE + lax.axis_index("c")
    tpw = nflat // _NW
    pltpu.sync_copy(params_hbm, params_v)

    def chunk_body(k, carry):
        base = pl.multiple_of(
            wid * jnp.int32(tpw) + k * jnp.int32(_CHUNK), _CHUNK)
        pltpu.sync_copy(ids_hbm.at[pl.ds(base, _CHUNK)], ids_v)
        gathers = []
        for i in range(_NHASH):
            am = _mod_nbuck(params_v[i, 0, :])
            cm = _mod_nbuck(am * jnp.int32(1024))
            bm = _mod_nbuck(params_v[i, 1, :])

            def hash_body(j, carry2):
                x = ids_v[pl.ds(j * jnp.int32(16), 16)]
                xh = lax.shift_right_logical(x, jnp.int32(10))
                xl = lax.bitwise_and(x, jnp.int32(1023))
                idx4_v[jnp.int32(i), pl.ds(j * jnp.int32(16), 16)] = (
                    _mod_nbuck(xh * cm + xl * am + bm))
                return carry2

            lax.fori_loop(jnp.int32(0), jnp.int32(_CHUNK // 16), hash_body,
                          jnp.int32(0))
            gathers.append(pltpu.async_copy(
                tables[i].at[idx4_v.at[jnp.int32(i)]],
                gath4_v.at[jnp.int32(i)], sem_g))
        outs = []
        for i in range(_NHASH):
            gathers[i].wait()
            outs.append(pltpu.async_copy(
                gath4_v.at[jnp.int32(i)],
                out_hbm.at[jnp.int32(i), pl.ds(base, _CHUNK), :], sem_o))
        for cp in outs:
            cp.wait()
        return carry

    lax.fori_loop(jnp.int32(0), jnp.int32(tpw // _CHUNK), chunk_body,
                  jnp.int32(0))


@functools.lru_cache(maxsize=None)
def _make_sc_gather(nflat):
    mesh = plsc.VectorSubcoreMesh(core_axis_name="c", subcore_axis_name="s")
    return pl.kernel(
        functools.partial(_sc_body, nflat),
        out_type=jax.ShapeDtypeStruct((_NHASH, nflat, _SUBD), jnp.float32),
        mesh=mesh,
        scratch_types=[
            pltpu.VMEM((_NHASH, 2, 16), jnp.int32),
            pltpu.VMEM((_CHUNK,), jnp.int32),
            pltpu.VMEM((_NHASH, _CHUNK), jnp.int32),
            pltpu.VMEM((_NHASH, _CHUNK, _SUBD), jnp.float32),
            pltpu.SemaphoreType.DMA,
            pltpu.SemaphoreType.DMA,
        ],
        compiler_params=pltpu.CompilerParams(use_tc_tiling_on_sc=False),
    )


def _mix_body(g_ref, w_ref, b_ref, o_ref):
    acc = jnp.dot(g_ref[0], w_ref[0], preferred_element_type=jnp.float32)
    for i in range(1, _NHASH):
        acc = acc + jnp.dot(g_ref[i], w_ref[i],
                            preferred_element_type=jnp.float32)
    o_ref[:, :] = acc + b_ref[0:1, :]


@functools.lru_cache(maxsize=None)
def _make_mix(npack, rows):
    return pl.pallas_call(
        _mix_body,
        grid=(npack // rows,),
        in_specs=[
            pl.BlockSpec((_NHASH, rows, _LANE),
                         lambda r: (jnp.int32(0), r, jnp.int32(0))),
            pl.BlockSpec((_NHASH, _LANE, _PACK * _EMBD),
                         lambda r: (jnp.int32(0), jnp.int32(0), jnp.int32(0))),
            pl.BlockSpec((1, _PACK * _EMBD),
                         lambda r: (jnp.int32(0), jnp.int32(0))),
        ],
        out_specs=pl.BlockSpec((rows, _PACK * _EMBD),
                               lambda r: (r, jnp.int32(0))),
        out_shape=jax.ShapeDtypeStruct((npack, _PACK * _EMBD), jnp.float32),
    )


def kernel(input_ids, table0, table1, table2, table3, W, b, hash_a, hash_b):
    bsz, seq = input_ids.shape
    nflat = bsz * seq
    flat = input_ids.reshape(-1).astype(jnp.int32)
    params = jnp.stack(
        [hash_a.astype(jnp.int32), hash_b.astype(jnp.int32)], axis=1)
    params = jnp.broadcast_to(params[:, :, None], (_NHASH, 2, 16))

    g = _make_sc_gather(nflat)(flat, params, table0, table1, table2, table3)
    g = g.reshape(_NHASH, nflat // _PACK, _LANE)

    w4 = W.astype(jnp.float32).T.reshape(_NHASH, _SUBD, _EMBD)
    eye = jnp.eye(_PACK, dtype=jnp.float32)
    wk = jax.vmap(lambda wi: jnp.kron(eye, wi))(w4)
    bp = jnp.tile(b.astype(jnp.float32), _PACK).reshape(1, _PACK * _EMBD)

    packed = _make_mix(nflat // _PACK, 1024)(g, wk, bp)
    out_dtype = jnp.result_type(table0.dtype, W.dtype, b.dtype)
    return packed.astype(out_dtype).reshape(bsz, seq, _EMBD)


# speedup vs baseline: 1.0212x; 1.0026x over previous
import functools

import jax
import jax.numpy as jnp
from jax import lax
from jax.experimental import pallas as pl
from jax.experimental.pallas import tpu as pltpu
from jax.experimental.pallas import tpu_sc as plsc

_NBUCK = 100000
_NHASH = 4
_SUBD = 16
_EMBD = 64
_PACK = 8
_LANE = _PACK * _SUBD
_NCORE = 2
_NSUB = 16
_NW = _NCORE * _NSUB
_CHUNK = 1024


def _mod_nbuck(t):
    nb = jnp.int32(_NBUCK)
    q = (t.astype(jnp.float32) * jnp.float32(1.0 / _NBUCK)).astype(jnp.int32)
    r = t - q * nb
    r = jnp.where(r < 0, r + nb, r)
    r = jnp.where(r >= nb, r - nb, r)
    return r


def _sc_body(nflat, ids_hbm, params_hbm, t0, t1, t2, t3, out_hbm,
             params_v, ids_v, idx4_v, gath4_v, sem_g, sem_o):
    tables = (t0, t1, t2, t3)
    wid = lax.axis_index("s") * _NCORE + lax.axis_index("c")
    tpw = nflat // _NW
    pltpu.sync_copy(params_hbm, params_v)

    def chunk_body(k, carry):
        base = pl.multiple_of(
            wid * jnp.int32(tpw) + k * jnp.int32(_CHUNK), _CHUNK)
        pltpu.sync_copy(ids_hbm.at[pl.ds(base, _CHUNK)], ids_v)
        gathers = []
        for i in range(_NHASH):
            am = _mod_nbuck(params_v[i, 0, :])
            cm = _mod_nbuck(am * jnp.int32(1024))
            bm = _mod_nbuck(params_v[i, 1, :])

            def hash_body(j, carry2):
                x = ids_v[pl.ds(j * jnp.int32(16), 16)]
                xh = lax.shift_right_logical(x, jnp.int32(10))
                xl = lax.bitwise_and(x, jnp.int32(1023))
                idx4_v[jnp.int32(i), pl.ds(j * jnp.int32(16), 16)] = (
                    _mod_nbuck(xh * cm + xl * am + bm))
                return carry2

            lax.fori_loop(jnp.int32(0), jnp.int32(_CHUNK // 16), hash_body,
                          jnp.int32(0))
            gathers.append(pltpu.async_copy(
                tables[i].at[idx4_v.at[jnp.int32(i)]],
                gath4_v.at[jnp.int32(i)], sem_g))
        outs = []
        for i in range(_NHASH):
            gathers[i].wait()
            outs.append(pltpu.async_copy(
                gath4_v.at[jnp.int32(i)],
                out_hbm.at[jnp.int32(i), pl.ds(base, _CHUNK), :], sem_o))
        for cp in outs:
            cp.wait()
        return carry

    lax.fori_loop(jnp.int32(0), jnp.int32(tpw // _CHUNK), chunk_body,
                  jnp.int32(0))


@functools.lru_cache(maxsize=None)
def _make_sc_gather(nflat):
    mesh = plsc.VectorSubcoreMesh(core_axis_name="c", subcore_axis_name="s")
    return pl.kernel(
        functools.partial(_sc_body, nflat),
        out_type=jax.ShapeDtypeStruct((_NHASH, nflat, _SUBD), jnp.float32),
        mesh=mesh,
        scratch_types=[
            pltpu.VMEM((_NHASH, 2, 16), jnp.int32),
            pltpu.VMEM((_CHUNK,), jnp.int32),
            pltpu.VMEM((_NHASH, _CHUNK), jnp.int32),
            pltpu.VMEM((_NHASH, _CHUNK, _SUBD), jnp.float32),
            pltpu.SemaphoreType.DMA,
            pltpu.SemaphoreType.DMA,
        ],
        compiler_params=pltpu.CompilerParams(use_tc_tiling_on_sc=False),
    )


def _mix_body(g_ref, w_ref, b_ref, o_ref):
    acc = jnp.dot(g_ref[0], w_ref[0], preferred_element_type=jnp.float32)
    for i in range(1, _NHASH):
        acc = acc + jnp.dot(g_ref[i], w_ref[i],
                            preferred_element_type=jnp.float32)
    o_ref[:, :] = acc + b_ref[0:1, :]


@functools.lru_cache(maxsize=None)
def _make_mix(npack, rows):
    return pl.pallas_call(
        _mix_body,
        grid=(npack // rows,),
        in_specs=[
            pl.BlockSpec((_NHASH, rows, _LANE),
                         lambda r: (jnp.int32(0), r, jnp.int32(0))),
            pl.BlockSpec((_NHASH, _LANE, _PACK * _EMBD),
                         lambda r: (jnp.int32(0), jnp.int32(0), jnp.int32(0))),
            pl.BlockSpec((1, _PACK * _EMBD),
                         lambda r: (jnp.int32(0), jnp.int32(0))),
        ],
        out_specs=pl.BlockSpec((rows, _PACK * _EMBD),
                               lambda r: (r, jnp.int32(0))),
        out_shape=jax.ShapeDtypeStruct((npack, _PACK * _EMBD), jnp.float32),
    )


def kernel(input_ids, table0, table1, table2, table3, W, b, hash_a, hash_b):
    bsz, seq = input_ids.shape
    nflat = bsz * seq
    flat = input_ids.reshape(-1).astype(jnp.int32)
    params = jnp.stack(
        [hash_a.astype(jnp.int32), hash_b.astype(jnp.int32)], axis=1)
    params = jnp.broadcast_to(params[:, :, None], (_NHASH, 2, 16))

    g = _make_sc_gather(nflat)(flat, params, table0, table1, table2, table3)
    g = g.reshape(_NHASH, nflat // _PACK, _LANE)

    w4 = W.astype(jnp.float32).T.reshape(_NHASH, _SUBD, _EMBD)
    eye = jnp.eye(_PACK, dtype=jnp.float32)
    wk = jax.vmap(lambda wi: jnp.kron(eye, wi))(w4)
    bp = jnp.tile(b.astype(jnp.float32), _PACK).reshape(1, _PACK * _EMBD)

    packed = _make_mix(nflat // _PACK, 2048)(g, wk, bp)
    out_dtype = jnp.result_type(table0.dtype, W.dtype, b.dtype)
    return packed.astype(out_dtype).reshape(bsz, seq, _EMBD)
